# Initial kernel scaffold; baseline (speedup 1.0000x reference)
import jax
import jax.numpy as jnp
from jax.experimental import pallas as pl


def kernel(x, w000, w001, w002, w003, w004, w005, w006, w007, w008, w009, w010, w011, w012, w013, w014, w015, w016, w017, w018, w019, w020, w021, w022, w023, w024, w025, w026, w027, w028, w029, w030, w031, w032, w033, w034, w035, w036, w037, w038, w039, w040, w041, w042, w043, w044, w045, w046, w047, w048, w049, w050, w051, w052, w053, w054, w055, w056, w057, w058, w059, w060, w061, w062, w063, w064, w065, w066, w067, w068, w069, w070, w071, w072, w073, w074, w075, w076, w077, w078, w079, w080, w081, w082, w083, w084, w085, w086, w087, w088, w089, w090, w091, w092, w093, w094, w095, w096, w097, w098, w099, w100, w101, w102, w103, w104, w105, w106, w107, w108, w109, w110, w111, w112, w113, w114, w115, w116, w117, w118, w119, w120, w121, w122, w123, w124, w125, w126, w127, w128, w129, w130, w131, w132, w133, w134, w135, w136, w137, w138, w139, w140, w141, w142, w143, w144, w145, w146, w147, w148, w149, w150, w151, w152, w153, w154, w155, w156, w157, w158, w159, w160, w161, w162, w163, w164, w165, w166, w167, w168, w169, w170, w171, w172, w173, w174, w175, w176, w177, w178, w179, w180, w181, w182, w183, w184, w185, w186, w187, w188, w189, w190, w191, w192, w193, w194, w195, w196, w197, w198, w199, w200, w201, w202, w203, w204, w205, w206, w207, w208, w209, w210, w211, w212, w213, w214, w215, w216, w217, w218, w219, w220, w221, w222, w223, w224, w225, w226, w227, w228, w229, w230, w231):
    raise NotImplementedError("write your pallas kernel here")


# speedup vs baseline: 1.2762x; 1.2762x over previous
import math
from functools import partial

import jax
import jax.numpy as jnp
from jax.experimental import pallas as pl
from jax.experimental.pallas import tpu as pltpu

_MIB = 1024 * 1024

_STAGES = (
    (1, 3, 1, 32, 16, 2),
    (6, 3, 2, 16, 24, 3),
    (6, 5, 2, 24, 48, 3),
    (6, 3, 2, 48, 88, 4),
    (6, 5, 1, 88, 120, 4),
    (6, 5, 2, 120, 208, 5),
    (6, 3, 1, 208, 352, 2),
)


def _silu(v):
    return v * jax.nn.sigmoid(v)


def _const_spec(shape):
    n = len(shape)
    return pl.BlockSpec(shape, lambda b: (0,) * n)


def _se_gate(pool, w1, b1, w2, b2):
    h = jnp.dot(pool, w1, preferred_element_type=jnp.float32) + b1
    h = _silu(h)
    g = jnp.dot(h, w2, preferred_element_type=jnp.float32) + b2
    return jax.nn.sigmoid(g)


def _dw_taps_s1(hb, wd_ref, k, H, W):
    acc = None
    for di in range(k):
        band = hb[di:di + H, :, :].astype(jnp.float32)
        for dj in range(k):
            t = di * k + dj
            wt = wd_ref[t:t + 1, :][None]
            c = band[:, dj:dj + W, :] * wt
            acc = c if acc is None else acc + c
    return acc


def _finish_block(x_skip, acc, bd_ref, w1, b1, w2, b2, wp_ref, bp_ref,
                  o_ref, Ho, Wo, Cexp, Cout):
    y = acc + bd_ref[...][None]
    y = _silu(y)
    pool = jnp.mean(y.reshape(Ho * Wo, Cexp), axis=0, keepdims=True)
    g = _se_gate(pool, w1[...], b1[...], w2[...], b2[...])
    yb = y.astype(jnp.bfloat16).reshape(Ho * Wo, Cexp) * g.astype(jnp.bfloat16)
    out = jnp.dot(yb, wp_ref[...], preferred_element_type=jnp.float32)
    out = out + bp_ref[...]
    if x_skip is not None:
        out = out + x_skip.astype(jnp.float32)
    o_ref[...] = out.astype(jnp.bfloat16).reshape(Ho, Wo, Cout)


def _blk_s1_body(*refs, k, H, W, Cin, Cexp, Cout, has_expand, has_skip, pad):
    i = 0
    x_ref = refs[i]; i += 1
    if has_expand:
        we_ref = refs[i]; be_ref = refs[i + 1]; i += 2
    wd_ref, bd_ref, w1, b1, w2, b2, wp_ref, bp_ref, o_ref = refs[i:i + 9]

    Hp, Wp = H + 2 * pad, W + 2 * pad
    x = x_ref[...]
    if has_expand:
        hq = jnp.dot(x.reshape(Hp * Wp, Cin), we_ref[...],
                     preferred_element_type=jnp.float32) + be_ref[...]
        hq = _silu(hq).reshape(Hp, Wp, Cexp)
        ri = jax.lax.broadcasted_iota(jnp.int32, (Hp, Wp, 1), 0)
        ci = jax.lax.broadcasted_iota(jnp.int32, (Hp, Wp, 1), 1)
        inside = ((ri >= pad) & (ri < pad + H)
                  & (ci >= pad) & (ci < pad + W))
        hq = jnp.where(inside, hq, 0.0)
        hb = hq.astype(jnp.bfloat16)
    else:
        hb = x
    acc = _dw_taps_s1(hb, wd_ref, k, H, W)
    skip = None
    if has_skip:
        skip = x[pad:pad + H, pad:pad + W, :].reshape(H * W, Cin)
    _finish_block(skip, acc, bd_ref, w1, b1, w2, b2, wp_ref, bp_ref,
                  o_ref, H, W, Cexp, Cout)


def _blk_s2_body(*refs, k, H, W, Ho, Wo, Cin, Cexp, Cout, pad):
    p_refs = refs[0:4]
    we_ref, be_ref, wd_ref, bd_ref, w1, b1, w2, b2, wp_ref, bp_ref, o_ref = refs[4:]
    planes = []
    for r in range(2):
        for s in range(2):
            pv = p_refs[2 * r + s][...]
            hs, ws = pv.shape[0], pv.shape[1]
            hq = jnp.dot(pv.reshape(hs * ws, Cin), we_ref[...],
                         preferred_element_type=jnp.float32) + be_ref[...]
            hq = _silu(hq).reshape(hs, ws, Cexp)
            ri = jax.lax.broadcasted_iota(jnp.int32, (hs, ws, 1), 0)
            ci = jax.lax.broadcasted_iota(jnp.int32, (hs, ws, 1), 1)
            inside = ((r + 2 * ri >= pad) & (r + 2 * ri < pad + H)
                      & (s + 2 * ci >= pad) & (s + 2 * ci < pad + W))
            hq = jnp.where(inside, hq, 0.0)
            planes.append(hq.astype(jnp.bfloat16))
    acc = None
    for r in range(2):
        for s in range(2):
            xr = planes[2 * r + s]
            na = (k - 1 - r) // 2 + 1
            nb = (k - 1 - s) // 2 + 1
            for a in range(na):
                band = xr[a:a + Ho, :, :].astype(jnp.float32)
                for b2_ in range(nb):
                    di, dj = 2 * a + r, 2 * b2_ + s
                    t = di * k + dj
                    wt = wd_ref[t:t + 1, :][None]
                    c = band[:, b2_:b2_ + Wo, :] * wt
                    acc = c if acc is None else acc + c
    _finish_block(None, acc, bd_ref, w1, b1, w2, b2, wp_ref, bp_ref,
                  o_ref, Ho, Wo, Cexp, Cout)


def _stem_body(*refs, H, W, Cout):
    (pt_ref, ws_ref, bs_ref, wd_ref, bd_ref,
     w1, b1, w2, b2, wp_ref, bp_ref, o_ref) = refs
    Hp, Wp = H + 2, W + 2
    Cstem = ws_ref.shape[1]
    p = pt_ref[...]
    h = jnp.dot(p.reshape(Hp * Wp, 27), ws_ref[...],
                preferred_element_type=jnp.float32) + bs_ref[...]
    h = _silu(h).reshape(Hp, Wp, Cstem)
    ri = jax.lax.broadcasted_iota(jnp.int32, (Hp, Wp, 1), 0)
    ci = jax.lax.broadcasted_iota(jnp.int32, (Hp, Wp, 1), 1)
    inside = (ri >= 1) & (ri < 1 + H) & (ci >= 1) & (ci < 1 + W)
    h = jnp.where(inside, h, 0.0)
    hb = h.astype(jnp.bfloat16)
    acc = _dw_taps_s1(hb, wd_ref, 3, H, W)
    _finish_block(None, acc, bd_ref, w1, b1, w2, b2, wp_ref, bp_ref,
                  o_ref, H, W, Cstem, Cout)


def _head_body(x_ref, wh_ref, bh_ref, wf_ref, bf_ref, lo_ref, fe_ref, *, M, C):
    xv = x_ref[...].reshape(M, C)
    h = jnp.dot(xv, wh_ref[...], preferred_element_type=jnp.float32)
    h = _silu(h + bh_ref[...])
    hb = h.astype(jnp.bfloat16).astype(jnp.float32)
    feat = jnp.mean(hb, axis=0, keepdims=True)
    lo = jnp.dot(feat, wf_ref[...], preferred_element_type=jnp.float32)
    lo_ref[...] = lo + bf_ref[...]
    fe_ref[...] = feat


def _params(step_bytes):
    return pltpu.CompilerParams(
        dimension_semantics=("parallel",),
        vmem_limit_bytes=int(min(48 * _MIB, max(32 * _MIB, 3 * int(step_bytes)))))


def _mbconv_s1(x, prm, *, k, expand, skip):
    B, H, W, Cin = x.shape
    pad = (k - 1) // 2
    xp = jnp.pad(x, ((0, 0), (pad, pad), (pad, pad), (0, 0)))
    Hp, Wp = H + 2 * pad, W + 2 * pad
    Cexp = prm["dw_w"].shape[1]
    Cout = prm["pj_w"].shape[1]
    args = [xp]
    specs = [pl.BlockSpec((None, Hp, Wp, Cin), lambda b: (b, 0, 0, 0))]
    if expand:
        args += [prm["ex_w"], prm["ex_b"]]
        specs += [_const_spec(prm["ex_w"].shape), _const_spec(prm["ex_b"].shape)]
    for key in ("dw_w", "dw_b", "se_w1", "se_b1", "se_w2", "se_b2",
                "pj_w", "pj_b"):
        args.append(prm[key])
        specs.append(_const_spec(prm[key].shape))
    wbytes = sum(int(a.size) * a.dtype.itemsize for a in args[1:])
    step = (Hp * Wp * Cin * 2 + wbytes + H * W * Cout * 2
            + Hp * Wp * Cexp * 2 + H * W * Cexp * 8)
    flops = 2 * B * (Hp * Wp * Cin * Cexp * int(expand)
                     + k * k * H * W * Cexp + H * W * Cexp * Cout)
    out = pl.pallas_call(
        partial(_blk_s1_body, k=k, H=H, W=W, Cin=Cin, Cexp=Cexp, Cout=Cout,
                has_expand=expand, has_skip=skip, pad=pad),
        grid=(B,),
        in_specs=specs,
        out_specs=pl.BlockSpec((None, H, W, Cout), lambda b: (b, 0, 0, 0)),
        out_shape=jax.ShapeDtypeStruct((B, H, W, Cout), jnp.bfloat16),
        compiler_params=_params(step),
        cost_estimate=pl.CostEstimate(
            flops=int(flops),
            transcendentals=int(B * (Hp * Wp + 2 * H * W) * Cexp),
            bytes_accessed=int(B * (Hp * Wp * Cin + H * W * Cout) * 2 + wbytes)),
    )(*args)
    return out


def _mbconv_s2(x, prm, *, k):
    B, H, W, Cin = x.shape
    pad = (k - 1) // 2
    xp = jnp.pad(x, ((0, 0), (pad, pad), (pad, pad), (0, 0)))
    Hp, Wp = H + 2 * pad, W + 2 * pad
    Ho = (Hp - k) // 2 + 1
    Wo = (Wp - k) // 2 + 1
    subs = [xp[:, r::2, s::2, :] for r in range(2) for s in range(2)]
    Cexp = prm["dw_w"].shape[1]
    Cout = prm["pj_w"].shape[1]
    args = list(subs)
    specs = [pl.BlockSpec((None,) + s.shape[1:], lambda b: (b, 0, 0, 0))
             for s in subs]
    for key in ("ex_w", "ex_b", "dw_w", "dw_b", "se_w1", "se_b1",
                "se_w2", "se_b2", "pj_w", "pj_b"):
        args.append(prm[key])
        specs.append(_const_spec(prm[key].shape))
    wbytes = sum(int(a.size) * a.dtype.itemsize for a in args[4:])
    sub_elems = sum(int(s.shape[1]) * int(s.shape[2]) for s in subs)
    step = (sub_elems * Cin * 2 + wbytes + Ho * Wo * Cout * 2
            + sub_elems * Cexp * 2 + Ho * Wo * Cexp * 8)
    flops = 2 * B * (sub_elems * Cin * Cexp + k * k * Ho * Wo * Cexp
                     + Ho * Wo * Cexp * Cout)
    out = pl.pallas_call(
        partial(_blk_s2_body, k=k, H=H, W=W, Ho=Ho, Wo=Wo,
                Cin=Cin, Cexp=Cexp, Cout=Cout, pad=pad),
        grid=(B,),
        in_specs=specs,
        out_specs=pl.BlockSpec((None, Ho, Wo, Cout), lambda b: (b, 0, 0, 0)),
        out_shape=jax.ShapeDtypeStruct((B, Ho, Wo, Cout), jnp.bfloat16),
        compiler_params=_params(step),
        cost_estimate=pl.CostEstimate(
            flops=int(flops),
            transcendentals=int(B * (sub_elems + 2 * Ho * Wo) * Cexp),
            bytes_accessed=int(B * (sub_elems * Cin + Ho * Wo * Cout) * 2
                               + wbytes)),
    )(*args)
    return out


def _stem_block(x_nchw, stem_w, stem_b, prm):
    B = x_nchw.shape[0]
    H = W = 112
    Hp, Wp = H + 2, W + 2
    x = jnp.transpose(x_nchw, (0, 2, 3, 1)).astype(jnp.bfloat16)
    xq = jnp.pad(x, ((0, 0), (3, 3), (3, 3), (0, 0)))
    cols = [xq[:, di:di + 2 * Hp:2, dj:dj + 2 * Wp:2, :]
            for di in range(3) for dj in range(3)]
    pt = jnp.concatenate(cols, axis=-1)
    Cstem = stem_w.shape[1]
    Cout = prm["pj_w"].shape[1]
    args = [pt, stem_w, stem_b]
    specs = [pl.BlockSpec((None, Hp, Wp, 27), lambda b: (b, 0, 0, 0)),
             _const_spec(stem_w.shape), _const_spec(stem_b.shape)]
    for key in ("dw_w", "dw_b", "se_w1", "se_b1", "se_w2", "se_b2",
                "pj_w", "pj_b"):
        args.append(prm[key])
        specs.append(_const_spec(prm[key].shape))
    wbytes = sum(int(a.size) * a.dtype.itemsize for a in args[1:])
    step = (Hp * Wp * 27 * 2 + wbytes + H * W * Cout * 2
            + Hp * Wp * Cstem * 2 + H * W * Cstem * 8)
    flops = 2 * B * (Hp * Wp * 27 * Cstem + 9 * H * W * Cstem
                     + H * W * Cstem * Cout)
    out = pl.pallas_call(
        partial(_stem_body, H=H, W=W, Cout=Cout),
        grid=(B,),
        in_specs=specs,
        out_specs=pl.BlockSpec((None, H, W, Cout), lambda b: (b, 0, 0, 0)),
        out_shape=jax.ShapeDtypeStruct((B, H, W, Cout), jnp.bfloat16),
        compiler_params=_params(step),
        cost_estimate=pl.CostEstimate(
            flops=int(flops),
            transcendentals=int(B * (Hp * Wp + 2 * H * W) * Cstem),
            bytes_accessed=int(B * (Hp * Wp * 27 + H * W * Cout) * 2 + wbytes)),
    )(*args)
    return out


def _head(x, head_w, head_b, fc_w, fc_b):
    B, H, W, C = x.shape
    M = H * W
    N = head_w.shape[1]
    ncls = fc_w.shape[1]
    wbytes = (int(head_w.size) * 2 + int(head_b.size) * 4
              + int(fc_w.size) * 4 + int(fc_b.size) * 4)
    step = M * C * 2 + wbytes + M * N * 6 + N * 4 + ncls * 4
    logits, feat = pl.pallas_call(
        partial(_head_body, M=M, C=C),
        grid=(B,),
        in_specs=[pl.BlockSpec((None, H, W, C), lambda b: (b, 0, 0, 0)),
                  _const_spec(head_w.shape), _const_spec(head_b.shape),
                  _const_spec(fc_w.shape), _const_spec(fc_b.shape)],
        out_specs=[pl.BlockSpec((None, 1, ncls), lambda b: (b, 0, 0)),
                   pl.BlockSpec((None, 1, N), lambda b: (b, 0, 0))],
        out_shape=(jax.ShapeDtypeStruct((B, 1, ncls), jnp.float32),
                   jax.ShapeDtypeStruct((B, 1, N), jnp.float32)),
        compiler_params=_params(step),
        cost_estimate=pl.CostEstimate(
            flops=int(2 * B * (M * C * N + N * ncls)),
            transcendentals=int(B * M * N),
            bytes_accessed=int(B * (M * C * 2 + N * 4 + ncls * 4) + wbytes)),
    )(x, head_w, head_b, fc_w, fc_b)
    return logits.reshape(B, ncls), feat.reshape(B, N)


def _unpack(ws):
    fc_b, fc_w, head_b, head_w = ws[0], ws[1], ws[2], ws[3]
    i = 4
    stages = []
    for (expand, _k, _s, _ci, _co, layers) in _STAGES:
        blocks = []
        for _li in range(layers):
            if expand == 1:
                dw_b, dw_w, pj_b, pj_w, b1, b2, w1, w2 = ws[i:i + 8]
                i += 8
                blk = dict(dw_w=dw_w, dw_b=dw_b, pj_w=pj_w, pj_b=pj_b,
                           se_w1=w1, se_b1=b1, se_w2=w2, se_b2=b2)
            else:
                dw_b, dw_w, ex_b, ex_w, pj_b, pj_w, b1, b2, w1, w2 = ws[i:i + 10]
                i += 10
                blk = dict(dw_w=dw_w, dw_b=dw_b, ex_w=ex_w, ex_b=ex_b,
                           pj_w=pj_w, pj_b=pj_b,
                           se_w1=w1, se_b1=b1, se_w2=w2, se_b2=b2)
            blocks.append(blk)
        stages.append(blocks)
    stem_b, stem_w = ws[i], ws[i + 1]
    return fc_b, fc_w, head_b, head_w, stages, stem_b, stem_w


def kernel(x, *ws):
    fc_b, fc_w, head_b, head_w, stages, stem_b, stem_w = _unpack(list(ws))

    h = _stem_block(x, stem_w, stem_b, stages[0][0])
    h = _mbconv_s1(h, stages[0][1], k=3, expand=False, skip=True)

    for si, (expand, k, stride, cin, cout, layers) in enumerate(_STAGES):
        if si == 0:
            continue
        for li in range(layers):
            prm = stages[si][li]
            if li == 0 and stride == 2:
                h = _mbconv_s2(h, prm, k=k)
            else:
                skip = li > 0
                h = _mbconv_s1(h, prm, k=k, expand=True, skip=skip)

    return _head(h, head_w, head_b, fc_w, fc_b)
